# baseline (speedup 1.0000x reference)
import jax
import jax.numpy as jnp
from jax import lax
from jax.experimental import pallas as pl
from jax.experimental.pallas import tpu as pltpu
from jax.experimental.pallas import tpu_sc as plsc

_B = 16384
_V = 100000
_NDIS = 26
_NCONT = 13
_D = 16

_NCORES = 2
_NSUB = 16
_NW = _NCORES * _NSUB
_BPW = _B // _NW
_NC = 64
_NCHUNK = _BPW // _NC
_NGRP = _NC // 16


def _fm_body(xtr, wfm, wlin, wc, out, xt_v, idxb_v, idx2_v, idx3_v, rows_v,
             lin_v, wc_v, wcb_v, out_v, sem_fm, sem_lin):
    wid = lax.axis_index("s") * _NCORES + lax.axis_index("c")
    pltpu.sync_copy(xtr.at[wid], xt_v)
    pltpu.sync_copy(wc, wc_v)

    lanes = lax.iota(jnp.int32, 16)
    zeros = jnp.zeros((16,), jnp.float32)

    for cf in range(_NCONT):
        row = wc_v[pl.ds(cf * _D, _D)]
        for d in range(_D):
            wcb_v[pl.ds((cf * _D + d) * 16, 16)] = zeros + row[d]

    @pl.loop(0, _BPW // 16)
    def _idx(g):
        off = pl.multiple_of(g * 16, 16)
        for f in range(_NDIS):
            xv = xt_v[pl.ds(f * _BPW + off, 16)]
            iv = xv.astype(jnp.int32)
            idxb_v[pl.ds(f * _BPW + off, 16)] = iv
            idx2_v[pl.ds(f * _BPW + off, 16)] = iv + f * _V

    cp_lin = pltpu.async_copy(wlin.at[idx2_v], lin_v, sem_lin)
    cp_lin.wait()

    @pl.loop(0, _NCHUNK)
    def _chunk(c):
        cbase = c * _NC

        @pl.loop(0, _NGRP)
        def _i3(g):
            off = pl.multiple_of(cbase + g * 16, 16)
            for f in range(_NDIS):
                bx = idxb_v[pl.ds(f * _BPW + off, 16)]
                for d in range(_D):
                    idx3_v[pl.ds((f * _D + d) * _NC + g * 16, 16)] = (
                        bx + (f * _D + d) * _V)

        cp_fm = pltpu.async_copy(wfm.at[idx3_v], rows_v, sem_fm)
        cp_fm.wait()

        @pl.loop(0, _NGRP)
        def _fm(g):
            off = pl.multiple_of(cbase + g * 16, 16)
            goff = g * 16

            linacc = lin_v[pl.ds(off, 16)]
            for f in range(1, _NDIS):
                linacc = linacc + lin_v[pl.ds(f * _BPW + off, 16)]
            for cf in range(_NCONT):
                linacc = linacc + xt_v[pl.ds((_NDIS + cf) * _BPW + off, 16)]

            css = jnp.zeros((16,), jnp.float32)
            cq = jnp.zeros((16,), jnp.float32)
            for h in range(2):
                d0 = h * 8
                s_acc = [jnp.zeros((16,), jnp.float32) for _ in range(8)]
                q_acc = [jnp.zeros((16,), jnp.float32) for _ in range(8)]
                for f in range(_NDIS):
                    xfv = xt_v[pl.ds(f * _BPW + off, 16)]
                    for dd in range(8):
                        rv = rows_v[
                            pl.ds((f * _D + d0 + dd) * _NC + goff, 16)]
                        t = rv * xfv
                        s_acc[dd] = s_acc[dd] + t
                        q_acc[dd] = q_acc[dd] + t * t
                for cf in range(_NCONT):
                    xcv = xt_v[pl.ds((_NDIS + cf) * _BPW + off, 16)]
                    for dd in range(8):
                        t = xcv * wcb_v[pl.ds((cf * _D + d0 + dd) * 16, 16)]
                        s_acc[dd] = s_acc[dd] + t
                        q_acc[dd] = q_acc[dd] + t * t
                for dd in range(8):
                    css = css + s_acc[dd] * s_acc[dd]
                    cq = cq + q_acc[dd]

            z = linacc + 0.5 * (css - cq)
            out_v[pl.ds(off, 16)] = 1.0 / (1.0 + jnp.exp(-z))

    base = pl.multiple_of(wid * _BPW, 8)
    pltpu.sync_copy(out_v, out.at[pl.ds(base, _BPW)])


_fm_call = pl.kernel(
    _fm_body,
    out_type=jax.ShapeDtypeStruct((_B,), jnp.float32),
    mesh=plsc.VectorSubcoreMesh(
        core_axis_name="c", subcore_axis_name="s",
        num_cores=_NCORES, num_subcores=_NSUB),
    scratch_types=[
        pltpu.VMEM(((_NDIS + _NCONT) * _BPW,), jnp.float32),
        pltpu.VMEM((_NDIS * _BPW,), jnp.int32),
        pltpu.VMEM((_NDIS * _BPW,), jnp.int32),
        pltpu.VMEM((_NDIS * _D * _NC,), jnp.int32),
        pltpu.VMEM((_NDIS * _D * _NC,), jnp.float32),
        pltpu.VMEM((_NDIS * _BPW,), jnp.float32),
        pltpu.VMEM((_NCONT * _D,), jnp.float32),
        pltpu.VMEM((_NCONT * _D * 16,), jnp.float32),
        pltpu.VMEM((_BPW,), jnp.float32),
        pltpu.SemaphoreType.DMA,
        pltpu.SemaphoreType.DMA,
    ],
    compiler_params=pltpu.CompilerParams(
        needs_layout_passes=False, use_tc_tiling_on_sc=False),
)


def kernel(X, W_lin, W_fm, W_cont):
    xtr = X.T.reshape(_NDIS + _NCONT, _NW, _BPW).transpose(1, 0, 2)
    xtr = xtr.reshape(_NW, (_NDIS + _NCONT) * _BPW)
    wfm_flat = W_fm.transpose(0, 2, 1).reshape(_NDIS * _D * _V)
    wlin_flat = W_lin.transpose(0, 2, 1).reshape(_NDIS * _V)
    out = _fm_call(xtr, wfm_flat, wlin_flat, W_cont.reshape(_NCONT * _D))
    return out.reshape(_B, 1)


# speedup vs baseline: 1.0187x; 1.0187x over previous
import jax
import jax.numpy as jnp
from jax import lax
from jax.experimental import pallas as pl
from jax.experimental.pallas import tpu as pltpu
from jax.experimental.pallas import tpu_sc as plsc

_B = 16384
_V = 100000
_NDIS = 26
_NCONT = 13
_D = 16

_NCORES = 2
_NSUB = 16
_NW = _NCORES * _NSUB
_BPW = _B // _NW
_NC = 64
_NCHUNK = _BPW // _NC
_NGRP = _NC // 16


def _fm_body(xtr, wfm, wlin, wc, out, xt_v, idxb_v, idx2_v, idx3_v, rows_v,
             lin_v, wc_v, wcb_v, out_v, sem_fm, sem_lin):
    wid = lax.axis_index("s") * _NCORES + lax.axis_index("c")
    pltpu.sync_copy(xtr.at[wid], xt_v)
    pltpu.sync_copy(wc, wc_v)

    lanes = lax.iota(jnp.int32, 16)
    zeros = jnp.zeros((16,), jnp.float32)

    for cf in range(_NCONT):
        row = wc_v[pl.ds(cf * _D, _D)]
        for d in range(_D):
            wcb_v[pl.ds((cf * _D + d) * 16, 16)] = zeros + row[d]

    @pl.loop(0, _BPW // 16)
    def _idx(g):
        off = pl.multiple_of(g * 16, 16)
        for f in range(_NDIS):
            xv = xt_v[pl.ds(f * _BPW + off, 16)]
            iv = xv.astype(jnp.int32)
            idxb_v[pl.ds(f * _BPW + off, 16)] = iv
            idx2_v[pl.ds(f * _BPW + off, 16)] = iv + f * _V

    cp_lin = pltpu.async_copy(wlin.at[idx2_v], lin_v, sem_lin)
    cp_lin.wait()

    @pl.loop(0, _NCHUNK)
    def _chunk(c):
        cbase = c * _NC

        @pl.loop(0, _NGRP)
        def _i3(g):
            off = pl.multiple_of(cbase + g * 16, 16)
            for f in range(_NDIS):
                bx = idxb_v[pl.ds(f * _BPW + off, 16)]
                for d in range(_D):
                    idx3_v[pl.ds((f * _D + d) * _NC + g * 16, 16)] = (
                        bx + (f * _D + d) * _V)

        _H = (_NDIS * _D * _NC) // 2
        cp_a = pltpu.async_copy(
            wfm.at[idx3_v.at[pl.ds(0, _H)]],
            rows_v.at[pl.ds(0, _H)], sem_fm)
        cp_b = pltpu.async_copy(
            wfm.at[idx3_v.at[pl.ds(_H, _H)]],
            rows_v.at[pl.ds(_H, _H)], sem_lin)
        cp_a.wait()
        cp_b.wait()

        @pl.loop(0, _NGRP)
        def _fm(g):
            off = pl.multiple_of(cbase + g * 16, 16)
            goff = g * 16

            linacc = lin_v[pl.ds(off, 16)]
            for f in range(1, _NDIS):
                linacc = linacc + lin_v[pl.ds(f * _BPW + off, 16)]
            for cf in range(_NCONT):
                linacc = linacc + xt_v[pl.ds((_NDIS + cf) * _BPW + off, 16)]

            css = jnp.zeros((16,), jnp.float32)
            cq = jnp.zeros((16,), jnp.float32)
            for h in range(2):
                d0 = h * 8
                s_acc = [jnp.zeros((16,), jnp.float32) for _ in range(8)]
                q_acc = [jnp.zeros((16,), jnp.float32) for _ in range(8)]
                for f in range(_NDIS):
                    xfv = xt_v[pl.ds(f * _BPW + off, 16)]
                    for dd in range(8):
                        rv = rows_v[
                            pl.ds((f * _D + d0 + dd) * _NC + goff, 16)]
                        t = rv * xfv
                        s_acc[dd] = s_acc[dd] + t
                        q_acc[dd] = q_acc[dd] + t * t
                for cf in range(_NCONT):
                    xcv = xt_v[pl.ds((_NDIS + cf) * _BPW + off, 16)]
                    for dd in range(8):
                        t = xcv * wcb_v[pl.ds((cf * _D + d0 + dd) * 16, 16)]
                        s_acc[dd] = s_acc[dd] + t
                        q_acc[dd] = q_acc[dd] + t * t
                for dd in range(8):
                    css = css + s_acc[dd] * s_acc[dd]
                    cq = cq + q_acc[dd]

            z = linacc + 0.5 * (css - cq)
            out_v[pl.ds(off, 16)] = 1.0 / (1.0 + jnp.exp(-z))

    base = pl.multiple_of(wid * _BPW, 8)
    pltpu.sync_copy(out_v, out.at[pl.ds(base, _BPW)])


_fm_call = pl.kernel(
    _fm_body,
    out_type=jax.ShapeDtypeStruct((_B,), jnp.float32),
    mesh=plsc.VectorSubcoreMesh(
        core_axis_name="c", subcore_axis_name="s",
        num_cores=_NCORES, num_subcores=_NSUB),
    scratch_types=[
        pltpu.VMEM(((_NDIS + _NCONT) * _BPW,), jnp.float32),
        pltpu.VMEM((_NDIS * _BPW,), jnp.int32),
        pltpu.VMEM((_NDIS * _BPW,), jnp.int32),
        pltpu.VMEM((_NDIS * _D * _NC,), jnp.int32),
        pltpu.VMEM((_NDIS * _D * _NC,), jnp.float32),
        pltpu.VMEM((_NDIS * _BPW,), jnp.float32),
        pltpu.VMEM((_NCONT * _D,), jnp.float32),
        pltpu.VMEM((_NCONT * _D * 16,), jnp.float32),
        pltpu.VMEM((_BPW,), jnp.float32),
        pltpu.SemaphoreType.DMA,
        pltpu.SemaphoreType.DMA,
    ],
    compiler_params=pltpu.CompilerParams(
        needs_layout_passes=False, use_tc_tiling_on_sc=False),
)


def kernel(X, W_lin, W_fm, W_cont):
    xtr = X.T.reshape(_NDIS + _NCONT, _NW, _BPW).transpose(1, 0, 2)
    xtr = xtr.reshape(_NW, (_NDIS + _NCONT) * _BPW)
    wfm_flat = W_fm.transpose(0, 2, 1).reshape(_NDIS * _D * _V)
    wlin_flat = W_lin.transpose(0, 2, 1).reshape(_NDIS * _V)
    out = _fm_call(xtr, wfm_flat, wlin_flat, W_cont.reshape(_NCONT * _D))
    return out.reshape(_B, 1)
